# Initial kernel scaffold; baseline (speedup 1.0000x reference)
import jax
import jax.numpy as jnp
from jax.experimental import pallas as pl


def kernel(h_x, t_x, params, h_edge_index, h_batch, t_edge_index, t_batch, b_edge_index, rels):
    raise NotImplementedError("write your pallas kernel here")


# speedup vs baseline: 17.1059x; 17.1059x over previous
import functools

import jax
import jax.numpy as jnp
from jax import lax
from jax.experimental import pallas as pl
from jax.experimental.pallas import tpu as pltpu
from jax.experimental.pallas import tpu_sc as plsc

N = 2048
NS = 2 * N
D = 128
G = 256
_NW = 32
_GRP = 64

_F32 = jnp.float32
_I32 = jnp.int32


def _hi(a, b):
    return jnp.matmul(a, b, precision=lax.Precision.HIGHEST,
                      preferred_element_type=_F32)


def _mm(a, b):
    return jnp.matmul(a, b, precision=lax.Precision.DEFAULT,
                      preferred_element_type=_F32)


def _bf(x):
    return x.astype(jnp.bfloat16).astype(_F32)


def _ln(x, g, b, eps=1e-5):
    mu = jnp.mean(x, -1, keepdims=True)
    v = jnp.mean((x - mu) ** 2, -1, keepdims=True)
    return (x - mu) / jnp.sqrt(v + eps) * g + b


def _elu(x):
    return jnp.where(x > 0, x, jnp.exp(jnp.minimum(x, 0.0)) - 1.0)


def _normed(x):
    n = jnp.sqrt(jnp.sum(x * x, -1, keepdims=True))
    return x / jnp.maximum(n, 1e-12)


@functools.cache
def _edge_kernel(E, H, weighted):
    EW = E // _NW
    NG = EW // _GRP
    ROWS = NS // 16
    OC = D // H
    NC16 = OC // 16
    mesh = plsc.VectorSubcoreMesh(core_axis_name="c", subcore_axis_name="s",
                                  num_cores=2, num_subcores=16)

    if weighted:
        out_type = [jax.ShapeDtypeStruct((2, NS, D), _F32),
                    jax.ShapeDtypeStruct((_NW, H * NS), _F32)]
        scratch = [
            pltpu.VMEM((NG, _GRP), _I32),
            pltpu.VMEM((NG, _GRP), _I32),
            pltpu.VMEM((H * NS,), _F32),
            pltpu.VMEM((H * NS,), _F32),
            pltpu.VMEM((H, 16), _F32),
            pltpu.VMEM((H * NS,), _F32),
            pltpu.VMEM((_GRP, D), _F32),
            pltpu.VMEM((H, _GRP), _F32),
            pltpu.VMEM_SHARED((NS, D), _F32),
            pltpu.SemaphoreType.DMA,
        ]

        @functools.partial(
            pl.kernel, out_type=out_type, mesh=mesh, scratch_types=scratch,
            compiler_params=pltpu.CompilerParams(needs_layout_passes=False))
        def k(s_hbm, d_hbm, als_hbm, ald_hbm, c_hbm, feat_hbm, zer_hbm,
              num_out, den_out,
              sidx_v, didx_v, als_v, ald_v, c_v, den_v, rows_v, w_v,
              num_sh, sem):
            cid = lax.axis_index("c")
            sid = lax.axis_index("s")
            wid = sid * 2 + cid
            pltpu.sync_copy(s_hbm.at[pl.ds(wid * NG, NG)], sidx_v)
            pltpu.sync_copy(d_hbm.at[pl.ds(wid * NG, NG)], didx_v)
            pltpu.sync_copy(als_hbm, als_v)
            pltpu.sync_copy(ald_hbm, ald_v)
            pltpu.sync_copy(c_hbm, c_v)
            pltpu.sync_copy(zer_hbm.at[pl.ds(sid * ROWS, ROWS)],
                            num_sh.at[pl.ds(sid * ROWS, ROWS)])

            def zden(i, carry):
                den_v[pl.ds(i * 16, 16)] = jnp.zeros((16,), _F32)
                return carry
            lax.fori_loop(0, (H * NS) // 16, zden, 0)
            plsc.subcore_barrier()

            def group(g, carry):
                pltpu.async_copy(feat_hbm.at[sidx_v.at[g]], rows_v, sem).wait()
                for j16 in range(_GRP // 16):
                    s16 = sidx_v[g, pl.ds(j16 * 16, 16)]
                    d16 = didx_v[g, pl.ds(j16 * 16, 16)]
                    for h in range(H):
                        hoff = jnp.full((16,), h * NS, _I32)
                        a_s = plsc.load_gather(als_v, [s16 + hoff])
                        a_d = plsc.load_gather(ald_v, [d16 + hoff])
                        e = a_s + a_d
                        e = jnp.where(e > 0, e, e * 0.2)
                        w = jnp.exp(e - c_v[h, :])
                        plsc.addupdate_scatter(den_v, [d16 + hoff], w)
                        w_v[h, pl.ds(j16 * 16, 16)] = w

                def rowscale(j, carry2):
                    for h in range(H):
                        wb = plsc.load_gather(
                            w_v, [jnp.full((16,), h, _I32),
                                  jnp.full((16,), j, _I32)])
                        for cc in range(NC16):
                            col = h * OC + cc * 16
                            rows_v[j, pl.ds(col, 16)] = (
                                rows_v[j, pl.ds(col, 16)] * wb)
                    return carry2
                lax.fori_loop(0, _GRP, rowscale, 0)
                pltpu.sync_copy(rows_v, num_sh.at[didx_v.at[g]], add=True)
                return carry
            lax.fori_loop(0, NG, group, 0)

            plsc.subcore_barrier()
            pltpu.sync_copy(num_sh.at[pl.ds(sid * ROWS, ROWS)],
                            num_out.at[cid, pl.ds(sid * ROWS, ROWS)])
            pltpu.sync_copy(den_v, den_out.at[wid])
        return k

    out_type = [jax.ShapeDtypeStruct((2, NS, D), _F32)]
    scratch = [
        pltpu.VMEM((NG, _GRP), _I32),
        pltpu.VMEM((NG, _GRP), _I32),
        pltpu.VMEM((_GRP, D), _F32),
        pltpu.VMEM_SHARED((NS, D), _F32),
        pltpu.SemaphoreType.DMA,
    ]

    @functools.partial(
        pl.kernel, out_type=out_type, mesh=mesh, scratch_types=scratch,
        compiler_params=pltpu.CompilerParams(needs_layout_passes=False))
    def ku(s_hbm, d_hbm, feat_hbm, zer_hbm, num_out,
           sidx_v, didx_v, rows_v, num_sh, sem):
        cid = lax.axis_index("c")
        sid = lax.axis_index("s")
        wid = sid * 2 + cid
        pltpu.sync_copy(s_hbm.at[pl.ds(wid * NG, NG)], sidx_v)
        pltpu.sync_copy(d_hbm.at[pl.ds(wid * NG, NG)], didx_v)
        pltpu.sync_copy(zer_hbm.at[pl.ds(sid * ROWS, ROWS)],
                        num_sh.at[pl.ds(sid * ROWS, ROWS)])
        plsc.subcore_barrier()

        def group(g, carry):
            pltpu.async_copy(feat_hbm.at[sidx_v.at[g]], rows_v, sem).wait()
            pltpu.sync_copy(rows_v, num_sh.at[didx_v.at[g]], add=True)
            return carry
        lax.fori_loop(0, NG, group, 0)

        plsc.subcore_barrier()
        pltpu.sync_copy(num_sh.at[pl.ds(sid * ROWS, ROWS)],
                        num_out.at[cid, pl.ds(sid * ROWS, ROWS)])
    return ku


def _edge_agg(s2d, d2d, als, ald, c, feat, zer, H):
    num_p, den_p = _edge_kernel(s2d.shape[0] * _GRP, H, True)(
        s2d, d2d, als, ald, c, feat, zer)
    return num_p, den_p


def _edge_sum(s2d, d2d, feat, zer):
    (num_p,) = _edge_kernel(s2d.shape[0] * _GRP, 1, False)(
        s2d, d2d, feat, zer)
    return num_p


def _tc(fn, out_shapes, *args):
    n_in = len(args)

    def body(*refs):
        vals = fn(*[r[...] for r in refs[:n_in]])
        if len(out_shapes) == 1:
            vals = (vals,)
        for r, v in zip(refs[n_in:], vals):
            r[...] = v

    res = pl.pallas_call(
        body,
        out_shape=[jax.ShapeDtypeStruct(s, _F32) for s in out_shapes],
    )(*args)
    return res[0] if len(out_shapes) == 1 else res


def _logits(src_t, dst_t, asrc, adst, H):
    oc = D // H
    als, ald, cs = [], [], []
    for h in range(H):
        a = jnp.sum(src_t[:, h * oc:(h + 1) * oc] * asrc[h][None, :], -1)
        b = jnp.sum(dst_t[:, h * oc:(h + 1) * oc] * adst[h][None, :], -1)
        als.append(a)
        ald.append(b)
        cs.append(jnp.full((16,), jnp.max(a) + jnp.max(b), _F32))
    return (jnp.stack(als), jnp.stack(ald), jnp.stack(cs))


def _gat_pre_same(x1, x2, W, asrc, adst, H):
    tbl = jnp.concatenate([_mm(x1, W), _mm(x2, W)], 0)
    als, ald, c = _logits(tbl, tbl, asrc, adst, H)
    return tbl, als, ald, c


def _gat_pre_inter(he, te, Wsrc, Wdst, asrc, adst):
    src_t = jnp.concatenate([_mm(he, Wsrc), _mm(te, Wsrc)], 0)
    dst_t = jnp.concatenate([_mm(he, Wdst), _mm(te, Wdst)], 0)
    als, ald, c = _logits(src_t, dst_t, asrc, adst, 1)
    return src_t, als, ald, c


def _gat_norm(num_p, den_p, bias, H):
    num = jnp.sum(num_p, 0)
    den = jnp.sum(den_p, 0).reshape(H, NS)
    oc = D // H
    cols = [jnp.broadcast_to(den[h][:, None], (NS, oc)) for h in range(H)]
    den_rep = jnp.concatenate(cols, 1)
    return num / (den_rep + 1e-16) + bias[None, :]


def _onehot(batch):
    return (batch[None, :] ==
            lax.broadcasted_iota(_I32, (G, N), 0)).astype(_F32)


def _sag_side(x, nb, attn, batch):
    oh = _onehot(batch)
    m = jnp.max(jnp.where(oh > 0, attn[None, :], -jnp.inf), 1)
    m = jnp.where(jnp.isfinite(m), m, 0.0)
    ex = jnp.exp(attn - _hi(m[None, :], oh)[0])
    den = _hi(oh, ex[:, None])[:, 0]
    score = ex / (_hi(den[None, :], oh)[0] + 1e-16)
    sag = x * score[:, None]
    return _hi(oh, sag)


def _stage_init(h_x, t_x, g, b):
    return _ln(h_x, g, b), _ln(t_x, g, b)


def _stage_fc_post(num_p, den_p, fc_b, pg_g, pg_b):
    out = _gat_norm(num_p, den_p, fc_b, 2)
    h = _ln(out[:N], pg_g, pg_b)
    t = _ln(out[N:], pg_g, pg_b)
    return _elu(h), _elu(t)


def _stage_red(num_ia, den_ia, ia_b, num_int, den_int, int_b, red_W, red_b):
    intra = _gat_norm(num_ia, den_ia, ia_b, 1)
    inter = _gat_norm(num_int, den_int, int_b, 1)
    h = _mm(jnp.concatenate([intra[:N], inter[:N]], 1), red_W) + red_b
    t = _mm(jnp.concatenate([intra[N:], inter[N:]], 1), red_W) + red_b
    return jnp.concatenate([h, t], 0), intra, inter


def _stage_sag(hf, nb_p, Wrel, Wroot, brel, h_batch, t_batch, nn_g, nn_b):
    nb = jnp.sum(nb_p, 0)
    attn = (_mm(nb, Wrel) + brel + _mm(hf, Wroot))[:, 0]
    h_pool = _sag_side(hf[:N], nb[:N], attn[:N], h_batch)
    t_pool = _sag_side(hf[N:], nb[N:], attn[N:], t_batch)
    nxt = _elu(_ln(hf, nn_g, nn_b))
    return h_pool, t_pool, nxt


def _projector(x, W1, b1, n1w, n1b, W2, b2, n2w, n2b):
    h = _mm(x, W1) + b1
    h = h * n1w / jnp.sqrt(1.0 + 1e-5) + n1b
    h = jnp.maximum(h, 0.0)
    h = _mm(h, W2) + b2
    h = h * n2w / jnp.sqrt(1.0 + 1e-5) + n2b
    return h


def _stage_contrast_pre(intra, inter, W1, b1, n1w, n1b, W2, b2, n2w, n2b):
    ip = _normed(_projector(intra, W1, b1, n1w, n1b, W2, b2, n2w, n2b))
    ep = _normed(_projector(inter, W1, b1, n1w, n1b, W2, b2, n2w, n2b))
    zh = _normed(jnp.concatenate([ip[:N], ep[:N]], 0))
    zt = _normed(jnp.concatenate([ip[N:], ep[N:]], 0))
    return ip, ep, zh, zt


def _lse(z):
    BLK = 512

    def body(zb_ref, z_ref, o_ref):
        i = pl.program_id(0)
        zb = zb_ref[...]
        zf = z_ref[...]
        s = lax.dot_general(zb, zf, (((1,), (1,)), ((), ())),
                            precision=lax.Precision.DEFAULT,
                            preferred_element_type=_F32) * 2.0
        rows = i * BLK + lax.broadcasted_iota(_I32, (BLK, NS), 0)
        cols = lax.broadcasted_iota(_I32, (BLK, NS), 1)
        s = s - jnp.where(rows == cols, 1e9, 0.0)
        m = jnp.max(s, 1)
        o_ref[...] = (m + jnp.log(jnp.sum(jnp.exp(s - m[:, None]), 1)))[:, None]

    return pl.pallas_call(
        body,
        grid=(NS // BLK,),
        in_specs=[pl.BlockSpec((BLK, D), lambda i: (i, 0)),
                  pl.BlockSpec((NS, D), lambda i: (0, 0))],
        out_specs=pl.BlockSpec((BLK, 1), lambda i: (i, 0)),
        out_shape=jax.ShapeDtypeStruct((NS, 1), _F32),
    )(z, z)


def _stage_final(lse_h, lse_t, zh, zt, ip, ep, hp1, hp2, tp1, tp2,
                 h_batch, t_batch, rels, cf_W, cf_b, cf_g, cf_bb,
                 fm_W1, fm_b1, fm_g1, fm_bb1, fm_W2, fm_b2, fm_g2, fm_bb2,
                 ca_wq, ca_wk, ca_bias, ca_a, kge3):
    pos_h = jnp.sum(_bf(zh[:N]) * _bf(zh[N:]), -1) * 2.0
    pos_t = jnp.sum(_bf(zt[:N]) * _bf(zt[N:]), -1) * 2.0
    closs = 0.5 * ((jnp.mean(lse_h) - jnp.mean(pos_h)) +
                   (jnp.mean(lse_t) - jnp.mean(pos_t)))

    oh_h = _onehot(h_batch)
    oh_t = _onehot(t_batch)
    hipg = _hi(oh_h, ip[:N])
    hepg = _hi(oh_h, ep[:N])
    tipg = _hi(oh_t, ip[N:])
    tepg = _hi(oh_t, ep[N:])
    h_c = jnp.maximum(_ln(_mm(jnp.concatenate([hipg, hepg], -1), cf_W)
                          + cf_b, cf_g, cf_bb), 0.0)
    t_c = jnp.maximum(_ln(_mm(jnp.concatenate([tipg, tepg], -1), cf_W)
                          + cf_b, cf_g, cf_bb), 0.0)

    def fusion(p, c):
        x = _mm(jnp.concatenate([p, c], -1), fm_W1) + fm_b1
        x = jnp.maximum(_ln(x, fm_g1, fm_bb1), 0.0)
        x = _mm(x, fm_W2) + fm_b2
        x = jnp.maximum(_ln(x, fm_g2, fm_bb2), 0.0)
        return x

    fh = fusion((hp1 + hp2) * 0.5, h_c)
    ft = fusion((tp1 + tp2) * 0.5, t_c)
    fh = _ln(fh, jnp.ones((D,), _F32), jnp.zeros((D,), _F32))
    ft = _ln(ft, jnp.ones((D,), _F32), jnp.zeros((D,), _F32))

    keys = _mm(fh, ca_wk)
    queries = _mm(ft, ca_wq)
    arows = []
    CH = 32
    for i0 in range(0, G, CH):
        e3 = queries[None, :, :] + keys[i0:i0 + CH, None, :] + ca_bias
        t3 = _bf(jnp.tanh(e3))
        arows.append(jnp.sum(t3 * _bf(ca_a)[None, None, :], -1))
    alpha = jnp.concatenate(arows, 0)

    fhn = _bf(_normed(fh))
    ftn = _bf(_normed(ft))
    B = _hi(alpha, ftn)
    A = lax.dot_general(fhn, B, (((0,), (0,)), ((), ())),
                        precision=lax.Precision.HIGHEST,
                        preferred_element_type=_F32)
    nrm = jnp.maximum(jnp.sqrt(jnp.sum(kge3 * kge3, (1, 2))), 1e-12)
    kn = _bf(kge3 / nrm[:, None, None])
    v = jnp.sum(kn * A[None, :, :], (1, 2))
    ohr = (rels[:, None] ==
           lax.broadcasted_iota(_I32, (G, 86), 1)).astype(_F32)
    scores = _hi(ohr, v[:, None])
    return scores, jnp.full((1, 1), 0.0, _F32) + closs


def kernel(h_x, t_x, params, h_edge_index, h_batch, t_edge_index, t_batch,
           b_edge_index, rels):
    P = params
    h_ei = h_edge_index.astype(_I32)
    t_ei = t_edge_index.astype(_I32)
    b_ei = b_edge_index.astype(_I32)
    h_batch = h_batch.astype(_I32)
    t_batch = t_batch.astype(_I32)
    rels = rels.astype(_I32)

    s_in = jnp.concatenate([h_ei[0], t_ei[0] + N]).reshape(-1, _GRP)
    d_in = jnp.concatenate([h_ei[1], t_ei[1] + N]).reshape(-1, _GRP)
    s_b = jnp.concatenate([b_ei[0], b_ei[1] + N]).reshape(-1, _GRP)
    d_b = jnp.concatenate([b_ei[1] + N, b_ei[0]]).reshape(-1, _GRP)
    zer = jnp.zeros((NS, D), _F32)

    h, t = _tc(_stage_init, [(N, D), (N, D)],
               h_x, t_x, P['init_g'], P['init_b'])

    pools = []
    intra = inter = None
    for Bp in P['blocks']:
        tbl, als, ald, c = _tc(
            functools.partial(_gat_pre_same, H=2),
            [(NS, D), (2, NS), (2, NS), (2, 16)],
            h, t, Bp['fc_W'], Bp['fc_asrc'], Bp['fc_adst'])
        num_p, den_p = _edge_agg(s_in, d_in, als.reshape(-1),
                                 ald.reshape(-1), c, tbl, zer, 2)
        he, te = _tc(_stage_fc_post, [(N, D), (N, D)],
                     num_p, den_p, Bp['fc_b'], Bp['pg_g'], Bp['pg_b'])

        tbl, als, ald, c = _tc(
            functools.partial(_gat_pre_same, H=1),
            [(NS, D), (1, NS), (1, NS), (1, 16)],
            he, te, Bp['ia_W'], Bp['ia_asrc'], Bp['ia_adst'])
        num_ia, den_ia = _edge_agg(s_in, d_in, als.reshape(-1),
                                   ald.reshape(-1), c, tbl, zer, 1)

        tbl, als, ald, c = _tc(
            _gat_pre_inter,
            [(NS, D), (1, NS), (1, NS), (1, 16)],
            he, te, Bp['int_Wsrc'], Bp['int_Wdst'],
            Bp['int_asrc'], Bp['int_adst'])
        num_int, den_int = _edge_agg(s_b, d_b, als.reshape(-1),
                                     ald.reshape(-1), c, tbl, zer, 1)

        hf, intra, inter = _tc(
            _stage_red, [(NS, D), (NS, D), (NS, D)],
            num_ia, den_ia, Bp['ia_b'], num_int, den_int, Bp['int_b'],
            Bp['red_W'], Bp['red_b'])

        nb_p = _edge_sum(s_in, d_in, hf, zer)
        h_pool, t_pool, nxt = _tc(
            _stage_sag, [(G, D), (G, D), (NS, D)],
            hf, nb_p, Bp['sag_Wrel'], Bp['sag_Wroot'], Bp['sag_brel'],
            h_batch, t_batch, Bp['nn_g'], Bp['nn_b'])
        pools.append((h_pool, t_pool))
        h, t = nxt[:N], nxt[N:]

    ip, ep, zh, zt = _tc(
        _stage_contrast_pre,
        [(NS, D), (NS, D), (NS, D), (NS, D)],
        intra, inter,
        P['cm_W1'], P['cm_b1'], P['cm_bn1w'], P['cm_bn1b'],
        P['cm_W2'], P['cm_b2'], P['cm_bn2w'], P['cm_bn2b'])

    lse_h = _lse(zh)
    lse_t = _lse(zt)

    kge3 = P['kge_rel'].reshape(86, D, D)
    scores, closs = _tc(
        _stage_final, [(G, 1), (1, 1)],
        lse_h, lse_t, zh, zt, ip, ep,
        pools[0][0], pools[1][0], pools[0][1], pools[1][1],
        h_batch, t_batch, rels,
        P['cf_W'], P['cf_b'], P['cf_g'], P['cf_bb'],
        P['fm_W1'], P['fm_b1'], P['fm_g1'], P['fm_bb1'],
        P['fm_W2'], P['fm_b2'], P['fm_g2'], P['fm_bb2'],
        P['ca_wq'], P['ca_wk'], P['ca_bias'], P['ca_a'], kge3)

    return scores.reshape(G), closs.reshape(())


# speedup vs baseline: 21.2177x; 1.2404x over previous
import functools

import jax
import jax.numpy as jnp
from jax import lax
from jax.experimental import pallas as pl
from jax.experimental.pallas import tpu as pltpu
from jax.experimental.pallas import tpu_sc as plsc

N = 2048
NS = 2 * N
D = 128
G = 256
_NW = 32
_GRP = 128

_F32 = jnp.float32
_I32 = jnp.int32


def _hi(a, b):
    return jnp.matmul(a, b, precision=lax.Precision.HIGHEST,
                      preferred_element_type=_F32)


def _mm(a, b):
    return jnp.matmul(a, b, precision=lax.Precision.DEFAULT,
                      preferred_element_type=_F32)


def _bf(x):
    return x.astype(jnp.bfloat16).astype(_F32)


def _ln(x, g, b, eps=1e-5):
    mu = jnp.mean(x, -1, keepdims=True)
    v = jnp.mean((x - mu) ** 2, -1, keepdims=True)
    return (x - mu) / jnp.sqrt(v + eps) * g + b


def _elu(x):
    return jnp.where(x > 0, x, jnp.exp(jnp.minimum(x, 0.0)) - 1.0)


def _normed(x):
    n = jnp.sqrt(jnp.sum(x * x, -1, keepdims=True))
    return x / jnp.maximum(n, 1e-12)


@functools.cache
def _edge_kernel(E, NT, H, weighted):
    GRP = 128
    EW = E // _NW
    NG = EW // GRP
    ROWS = NT // 16
    OC = D // H
    NC16 = OC // 16
    mesh = plsc.VectorSubcoreMesh(core_axis_name="c", subcore_axis_name="s",
                                  num_cores=2, num_subcores=16)

    out_type = [jax.ShapeDtypeStruct((2, NT, D), _F32)]
    scratch = [
        pltpu.VMEM((NG, GRP), _I32),
        pltpu.VMEM((NG, GRP), _I32),
        pltpu.VMEM((2, GRP, D), _F32),
        pltpu.VMEM_SHARED((NT, D), _F32),
        pltpu.SemaphoreType.DMA,
        pltpu.SemaphoreType.DMA,
        pltpu.SemaphoreType.DMA,
        pltpu.SemaphoreType.DMA,
    ]
    if weighted:
        out_type.append(jax.ShapeDtypeStruct((_NW, H * NT), _F32))
        scratch += [
            pltpu.VMEM((H * NT,), _F32),
            pltpu.VMEM((H * NT,), _F32),
            pltpu.VMEM((H, 16), _F32),
            pltpu.VMEM((H * NT,), _F32),
            pltpu.VMEM((H, GRP), _F32),
        ]

    def body(*refs):
        if weighted:
            (s_hbm, d_hbm, als_hbm, ald_hbm, c_hbm, feat_hbm, zer_hbm,
             num_out, den_out, sidx_v, didx_v, rows_v, num_sh,
             sg0, sg1, ss0, ss1,
             als_v, ald_v, c_v, den_v, w_v) = refs
        else:
            (s_hbm, d_hbm, feat_hbm, zer_hbm, num_out,
             sidx_v, didx_v, rows_v, num_sh,
             sg0, sg1, ss0, ss1) = refs
        sems_g = (sg0, sg1)
        sems_s = (ss0, ss1)
        cid = lax.axis_index("c")
        sid = lax.axis_index("s")
        wid = sid * 2 + cid
        pltpu.sync_copy(s_hbm.at[wid], sidx_v)
        pltpu.sync_copy(d_hbm.at[wid], didx_v)
        if weighted:
            pltpu.sync_copy(als_hbm, als_v)
            pltpu.sync_copy(ald_hbm, ald_v)
            pltpu.sync_copy(c_hbm, c_v)

            def zden(i, carry):
                den_v[pl.ds(i * 16, 16)] = jnp.zeros((16,), _F32)
                return carry
            lax.fori_loop(0, (H * NT) // 16, zden, 0)
        pltpu.sync_copy(zer_hbm.at[pl.ds(sid * ROWS, ROWS)],
                        num_sh.at[pl.ds(sid * ROWS, ROWS)])
        plsc.subcore_barrier()

        gd = [None] * NG
        sd = [None] * NG
        gd[0] = pltpu.async_copy(feat_hbm.at[sidx_v.at[0]],
                                 rows_v.at[0], sems_g[0])
        for g in range(NG):
            b = g % 2
            if weighted:
                for j16 in range(GRP // 16):
                    s16 = sidx_v[g, pl.ds(j16 * 16, 16)]
                    d16 = didx_v[g, pl.ds(j16 * 16, 16)]
                    for h in range(H):
                        hoff = jnp.full((16,), h * NT, _I32)
                        a_s = plsc.load_gather(als_v, [s16 + hoff])
                        a_d = plsc.load_gather(ald_v, [d16 + hoff])
                        e = a_s + a_d
                        e = jnp.where(e > 0, e, e * 0.2)
                        w = jnp.exp(e - c_v[h, :])
                        plsc.addupdate_scatter(den_v, [d16 + hoff], w)
                        w_v[h, pl.ds(j16 * 16, 16)] = w
            if g + 1 < NG:
                if g >= 1:
                    sd[g - 1].wait()
                gd[g + 1] = pltpu.async_copy(
                    feat_hbm.at[sidx_v.at[g + 1]],
                    rows_v.at[(g + 1) % 2], sems_g[(g + 1) % 2])
            gd[g].wait()
            if weighted:
                @plsc.parallel_loop(0, GRP, unroll=4)
                def rowscale(j):
                    for h in range(H):
                        wb = plsc.load_gather(
                            w_v, [jnp.full((16,), h, _I32),
                                  jnp.full((16,), j, _I32)])
                        for cc in range(NC16):
                            col = h * OC + cc * 16
                            rows_v[b, j, pl.ds(col, 16)] = (
                                rows_v[b, j, pl.ds(col, 16)] * wb)
            sd[g] = pltpu.async_copy(rows_v.at[b],
                                     num_sh.at[didx_v.at[g]],
                                     sems_s[b], add=True)
        sd[NG - 1].wait()
        if NG >= 2:
            sd[NG - 2].wait()

        plsc.subcore_barrier()
        pltpu.sync_copy(num_sh.at[pl.ds(sid * ROWS, ROWS)],
                        num_out.at[cid, pl.ds(sid * ROWS, ROWS)])
        if weighted:
            pltpu.sync_copy(den_v, den_out.at[wid])

    return functools.partial(
        pl.kernel, out_type=out_type, mesh=mesh, scratch_types=scratch,
        compiler_params=pltpu.CompilerParams(needs_layout_passes=False),
    )(body)


def _edge_agg(s2d, d2d, als, ald, c, feat, zer, H):
    s3 = s2d.reshape(_NW, -1, _GRP)
    d3 = d2d.reshape(_NW, -1, _GRP)
    num_p, den_p = _edge_kernel(s2d.size, feat.shape[0], H, True)(
        s3, d3, als, ald, c, feat, zer)
    return num_p, den_p


def _edge_sum(s2d, d2d, feat, zer):
    s3 = s2d.reshape(_NW, -1, _GRP)
    d3 = d2d.reshape(_NW, -1, _GRP)
    (num_p,) = _edge_kernel(s2d.size, feat.shape[0], 1, False)(
        s3, d3, feat, zer)
    return num_p


def _tc(fn, out_shapes, *args):
    n_in = len(args)

    def body(*refs):
        vals = fn(*[r[...] for r in refs[:n_in]])
        if len(out_shapes) == 1:
            vals = (vals,)
        for r, v in zip(refs[n_in:], vals):
            r[...] = v

    res = pl.pallas_call(
        body,
        out_shape=[jax.ShapeDtypeStruct(s, _F32) for s in out_shapes],
    )(*args)
    return res[0] if len(out_shapes) == 1 else res


def _logits(src_t, dst_t, asrc, adst, H):
    oc = D // H
    als, ald, cs = [], [], []
    for h in range(H):
        a = jnp.sum(src_t[:, h * oc:(h + 1) * oc] * asrc[h][None, :], -1)
        b = jnp.sum(dst_t[:, h * oc:(h + 1) * oc] * adst[h][None, :], -1)
        als.append(a)
        ald.append(b)
        cs.append(jnp.full((16,), jnp.max(a) + jnp.max(b), _F32))
    return (jnp.stack(als), jnp.stack(ald), jnp.stack(cs))


def _gat_pre_same(x1, x2, W, asrc, adst, H):
    tbl = jnp.concatenate([_mm(x1, W), _mm(x2, W)], 0)
    als, ald, c = _logits(tbl, tbl, asrc, adst, H)
    return tbl, als, ald, c


def _gat_pre_merged(he, te, Wia, ia_asrc, ia_adst, Wsrc, Wdst, asrc, adst):
    ia_t = jnp.concatenate([_mm(he, Wia), _mm(te, Wia)], 0)
    src_t = jnp.concatenate([_mm(he, Wsrc), _mm(te, Wsrc)], 0)
    dst_t = jnp.concatenate([_mm(he, Wdst), _mm(te, Wdst)], 0)
    als_i, ald_i, _ = _logits(ia_t, ia_t, ia_asrc, ia_adst, 1)
    als_e, ald_e, _ = _logits(src_t, dst_t, asrc, adst, 1)
    als = jnp.concatenate([als_i, als_e], 1)
    ald = jnp.concatenate([ald_i, ald_e], 1)
    c = jnp.max(als) + jnp.max(ald)
    return (jnp.concatenate([ia_t, src_t], 0), als, ald,
            jnp.full((1, 16), c, _F32))


def _gat_norm(num_p, den_p, bias, H):
    num = jnp.sum(num_p, 0)
    nt = num.shape[0]
    den = jnp.sum(den_p, 0).reshape(H, nt)
    oc = D // H
    cols = [jnp.broadcast_to(den[h][:, None], (nt, oc)) for h in range(H)]
    den_rep = jnp.concatenate(cols, 1)
    return num / (den_rep + 1e-16) + bias[None, :]


def _onehot(batch):
    return (batch[None, :] ==
            lax.broadcasted_iota(_I32, (G, N), 0)).astype(_F32)


def _sag_side(x, nb, attn, batch):
    oh = _onehot(batch)
    m = jnp.max(jnp.where(oh > 0, attn[None, :], -jnp.inf), 1)
    m = jnp.where(jnp.isfinite(m), m, 0.0)
    ex = jnp.exp(attn - _hi(m[None, :], oh)[0])
    den = _hi(oh, ex[:, None])[:, 0]
    score = ex / (_hi(den[None, :], oh)[0] + 1e-16)
    sag = x * score[:, None]
    return _hi(oh, sag)


def _stage_init(h_x, t_x, g, b):
    return _ln(h_x, g, b), _ln(t_x, g, b)


def _stage_fc_post(num_p, den_p, fc_b, pg_g, pg_b):
    out = _gat_norm(num_p, den_p, fc_b, 2)
    h = _ln(out[:N], pg_g, pg_b)
    t = _ln(out[N:], pg_g, pg_b)
    return _elu(h), _elu(t)


def _stage_red(num_m, den_m, ia_b, int_b, red_W, red_b):
    both = _gat_norm(num_m, den_m, jnp.zeros((D,), _F32), 1)
    intra = both[:NS] + ia_b[None, :]
    inter = both[NS:] + int_b[None, :]
    h = _mm(jnp.concatenate([intra[:N], inter[:N]], 1), red_W) + red_b
    t = _mm(jnp.concatenate([intra[N:], inter[N:]], 1), red_W) + red_b
    return jnp.concatenate([h, t], 0), intra, inter


def _stage_sag(hf, nb_p, Wrel, Wroot, brel, h_batch, t_batch, nn_g, nn_b):
    nb = jnp.sum(nb_p, 0)
    attn = (_mm(nb, Wrel) + brel + _mm(hf, Wroot))[:, 0]
    h_pool = _sag_side(hf[:N], nb[:N], attn[:N], h_batch)
    t_pool = _sag_side(hf[N:], nb[N:], attn[N:], t_batch)
    nxt = _elu(_ln(hf, nn_g, nn_b))
    return h_pool, t_pool, nxt


def _projector(x, W1, b1, n1w, n1b, W2, b2, n2w, n2b):
    h = _mm(x, W1) + b1
    h = h * n1w / jnp.sqrt(1.0 + 1e-5) + n1b
    h = jnp.maximum(h, 0.0)
    h = _mm(h, W2) + b2
    h = h * n2w / jnp.sqrt(1.0 + 1e-5) + n2b
    return h


def _stage_contrast_pre(intra, inter, W1, b1, n1w, n1b, W2, b2, n2w, n2b):
    ip = _normed(_projector(intra, W1, b1, n1w, n1b, W2, b2, n2w, n2b))
    ep = _normed(_projector(inter, W1, b1, n1w, n1b, W2, b2, n2w, n2b))
    zh = _normed(jnp.concatenate([ip[:N], ep[:N]], 0))
    zt = _normed(jnp.concatenate([ip[N:], ep[N:]], 0))
    return ip, ep, zh, zt


def _lse(z):
    BLK = 512

    def body(zb_ref, z_ref, o_ref):
        i = pl.program_id(0)
        zb = zb_ref[...]
        zf = z_ref[...]
        s = lax.dot_general(zb, zf, (((1,), (1,)), ((), ())),
                            precision=lax.Precision.DEFAULT,
                            preferred_element_type=_F32) * 2.0
        rows = i * BLK + lax.broadcasted_iota(_I32, (BLK, NS), 0)
        cols = lax.broadcasted_iota(_I32, (BLK, NS), 1)
        s = s - jnp.where(rows == cols, 1e9, 0.0)
        m = jnp.max(s, 1)
        o_ref[...] = (m + jnp.log(jnp.sum(jnp.exp(s - m[:, None]), 1)))[:, None]

    return pl.pallas_call(
        body,
        grid=(NS // BLK,),
        in_specs=[pl.BlockSpec((BLK, D), lambda i: (i, 0)),
                  pl.BlockSpec((NS, D), lambda i: (0, 0))],
        out_specs=pl.BlockSpec((BLK, 1), lambda i: (i, 0)),
        out_shape=jax.ShapeDtypeStruct((NS, 1), _F32),
    )(z, z)


def _stage_final(lse_h, lse_t, zh, zt, ip, ep, hp1, hp2, tp1, tp2,
                 h_batch, t_batch, rels, cf_W, cf_b, cf_g, cf_bb,
                 fm_W1, fm_b1, fm_g1, fm_bb1, fm_W2, fm_b2, fm_g2, fm_bb2,
                 ca_wq, ca_wk, ca_bias, ca_a, kge3):
    pos_h = jnp.sum(_bf(zh[:N]) * _bf(zh[N:]), -1) * 2.0
    pos_t = jnp.sum(_bf(zt[:N]) * _bf(zt[N:]), -1) * 2.0
    closs = 0.5 * ((jnp.mean(lse_h) - jnp.mean(pos_h)) +
                   (jnp.mean(lse_t) - jnp.mean(pos_t)))

    oh_h = _onehot(h_batch)
    oh_t = _onehot(t_batch)
    hipg = _hi(oh_h, ip[:N])
    hepg = _hi(oh_h, ep[:N])
    tipg = _hi(oh_t, ip[N:])
    tepg = _hi(oh_t, ep[N:])
    h_c = jnp.maximum(_ln(_mm(jnp.concatenate([hipg, hepg], -1), cf_W)
                          + cf_b, cf_g, cf_bb), 0.0)
    t_c = jnp.maximum(_ln(_mm(jnp.concatenate([tipg, tepg], -1), cf_W)
                          + cf_b, cf_g, cf_bb), 0.0)

    def fusion(p, c):
        x = _mm(jnp.concatenate([p, c], -1), fm_W1) + fm_b1
        x = jnp.maximum(_ln(x, fm_g1, fm_bb1), 0.0)
        x = _mm(x, fm_W2) + fm_b2
        x = jnp.maximum(_ln(x, fm_g2, fm_bb2), 0.0)
        return x

    fh = fusion((hp1 + hp2) * 0.5, h_c)
    ft = fusion((tp1 + tp2) * 0.5, t_c)
    fh = _ln(fh, jnp.ones((D,), _F32), jnp.zeros((D,), _F32))
    ft = _ln(ft, jnp.ones((D,), _F32), jnp.zeros((D,), _F32))

    keys = _mm(fh, ca_wk)
    queries = _mm(ft, ca_wq)
    arows = []
    CH = 32
    for i0 in range(0, G, CH):
        e3 = queries[None, :, :] + keys[i0:i0 + CH, None, :] + ca_bias
        t3 = _bf(jnp.tanh(e3))
        arows.append(jnp.sum(t3 * _bf(ca_a)[None, None, :], -1))
    alpha = jnp.concatenate(arows, 0)

    fhn = _bf(_normed(fh))
    ftn = _bf(_normed(ft))
    B = _hi(alpha, ftn)
    A = lax.dot_general(fhn, B, (((0,), (0,)), ((), ())),
                        precision=lax.Precision.HIGHEST,
                        preferred_element_type=_F32)
    nrm = jnp.maximum(jnp.sqrt(jnp.sum(kge3 * kge3, (1, 2))), 1e-12)
    kn = _bf(kge3 / nrm[:, None, None])
    v = jnp.sum(kn * A[None, :, :], (1, 2))
    ohr = (rels[:, None] ==
           lax.broadcasted_iota(_I32, (G, 86), 1)).astype(_F32)
    scores = _hi(ohr, v[:, None])
    return scores, jnp.full((1, 1), 0.0, _F32) + closs


def kernel(h_x, t_x, params, h_edge_index, h_batch, t_edge_index, t_batch,
           b_edge_index, rels):
    P = params
    h_ei = h_edge_index.astype(_I32)
    t_ei = t_edge_index.astype(_I32)
    b_ei = b_edge_index.astype(_I32)
    h_batch = h_batch.astype(_I32)
    t_batch = t_batch.astype(_I32)
    rels = rels.astype(_I32)

    s_in = jnp.concatenate([h_ei[0], t_ei[0] + N]).reshape(-1, _GRP)
    d_in = jnp.concatenate([h_ei[1], t_ei[1] + N]).reshape(-1, _GRP)
    s_b = jnp.concatenate([b_ei[0], b_ei[1] + N])
    d_b = jnp.concatenate([b_ei[1] + N, b_ei[0]])
    s_m = jnp.concatenate([s_in.reshape(-1), s_b + NS]).reshape(-1, _GRP)
    d_m = jnp.concatenate([d_in.reshape(-1), d_b + NS]).reshape(-1, _GRP)
    zer = jnp.zeros((NS, D), _F32)
    zer2 = jnp.zeros((2 * NS, D), _F32)

    h, t = _tc(_stage_init, [(N, D), (N, D)],
               h_x, t_x, P['init_g'], P['init_b'])

    pools = []
    intra = inter = None
    for Bp in P['blocks']:
        tbl, als, ald, c = _tc(
            functools.partial(_gat_pre_same, H=2),
            [(NS, D), (2, NS), (2, NS), (2, 16)],
            h, t, Bp['fc_W'], Bp['fc_asrc'], Bp['fc_adst'])
        num_p, den_p = _edge_agg(s_in, d_in, als.reshape(-1),
                                 ald.reshape(-1), c, tbl, zer, 2)
        he, te = _tc(_stage_fc_post, [(N, D), (N, D)],
                     num_p, den_p, Bp['fc_b'], Bp['pg_g'], Bp['pg_b'])

        tbl, als, ald, c = _tc(
            _gat_pre_merged,
            [(2 * NS, D), (1, 2 * NS), (1, 2 * NS), (1, 16)],
            he, te, Bp['ia_W'], Bp['ia_asrc'], Bp['ia_adst'],
            Bp['int_Wsrc'], Bp['int_Wdst'],
            Bp['int_asrc'], Bp['int_adst'])
        num_m, den_m = _edge_agg(s_m, d_m, als.reshape(-1),
                                 ald.reshape(-1), c, tbl, zer2, 1)

        hf, intra, inter = _tc(
            _stage_red, [(NS, D), (NS, D), (NS, D)],
            num_m, den_m, Bp['ia_b'], Bp['int_b'],
            Bp['red_W'], Bp['red_b'])

        nb_p = _edge_sum(s_in, d_in, hf, zer)
        h_pool, t_pool, nxt = _tc(
            _stage_sag, [(G, D), (G, D), (NS, D)],
            hf, nb_p, Bp['sag_Wrel'], Bp['sag_Wroot'], Bp['sag_brel'],
            h_batch, t_batch, Bp['nn_g'], Bp['nn_b'])
        pools.append((h_pool, t_pool))
        h, t = nxt[:N], nxt[N:]

    ip, ep, zh, zt = _tc(
        _stage_contrast_pre,
        [(NS, D), (NS, D), (NS, D), (NS, D)],
        intra, inter,
        P['cm_W1'], P['cm_b1'], P['cm_bn1w'], P['cm_bn1b'],
        P['cm_W2'], P['cm_b2'], P['cm_bn2w'], P['cm_bn2b'])

    lse_h = _lse(zh)
    lse_t = _lse(zt)

    kge3 = P['kge_rel'].reshape(86, D, D)
    scores, closs = _tc(
        _stage_final, [(G, 1), (1, 1)],
        lse_h, lse_t, zh, zt, ip, ep,
        pools[0][0], pools[1][0], pools[0][1], pools[1][1],
        h_batch, t_batch, rels,
        P['cf_W'], P['cf_b'], P['cf_g'], P['cf_bb'],
        P['fm_W1'], P['fm_b1'], P['fm_g1'], P['fm_bb1'],
        P['fm_W2'], P['fm_b2'], P['fm_g2'], P['fm_bb2'],
        P['ca_wq'], P['ca_wk'], P['ca_bias'], P['ca_a'], kge3)

    return scores.reshape(G), closs.reshape(())


# speedup vs baseline: 22.6278x; 1.0665x over previous
import functools

import jax
import jax.numpy as jnp
from jax import lax
from jax.experimental import pallas as pl
from jax.experimental.pallas import tpu as pltpu
from jax.experimental.pallas import tpu_sc as plsc

N = 2048
NS = 2 * N
D = 128
G = 256
_NW = 32
_GRP = 128

_F32 = jnp.float32
_I32 = jnp.int32


def _hi(a, b):
    return jnp.matmul(a, b, precision=lax.Precision.HIGHEST,
                      preferred_element_type=_F32)


def _mm(a, b):
    return jnp.matmul(a, b, precision=lax.Precision.DEFAULT,
                      preferred_element_type=_F32)


def _bf(x):
    return x.astype(jnp.bfloat16).astype(_F32)


def _ln(x, g, b, eps=1e-5):
    mu = jnp.mean(x, -1, keepdims=True)
    v = jnp.mean((x - mu) ** 2, -1, keepdims=True)
    return (x - mu) / jnp.sqrt(v + eps) * g + b


def _elu(x):
    return jnp.where(x > 0, x, jnp.exp(jnp.minimum(x, 0.0)) - 1.0)


def _normed(x):
    n = jnp.sqrt(jnp.sum(x * x, -1, keepdims=True))
    return x / jnp.maximum(n, 1e-12)


@functools.cache
def _edge_kernel(E, NT, H, weighted):
    GRP = 128
    EW = E // _NW
    NG = EW // GRP
    ROWS = NT // 16
    OC = D // H
    NC16 = OC // 16
    mesh = plsc.VectorSubcoreMesh(core_axis_name="c", subcore_axis_name="s",
                                  num_cores=2, num_subcores=16)

    out_type = [jax.ShapeDtypeStruct((2, NT, D), _F32)]
    scratch = [
        pltpu.VMEM((NG, GRP), _I32),
        pltpu.VMEM((NG, GRP), _I32),
        pltpu.VMEM((2, GRP, D), _F32),
        pltpu.VMEM_SHARED((NT, D), _F32),
        pltpu.SemaphoreType.DMA,
        pltpu.SemaphoreType.DMA,
        pltpu.SemaphoreType.DMA,
        pltpu.SemaphoreType.DMA,
    ]
    if weighted:
        out_type.append(jax.ShapeDtypeStruct((_NW, H * NT), _F32))
        scratch += [
            pltpu.VMEM((H * NT,), _F32),
            pltpu.VMEM((H * NT,), _F32),
            pltpu.VMEM((H, 16), _F32),
            pltpu.VMEM((H * NT,), _F32),
            pltpu.VMEM((H, GRP), _F32),
        ]

    def body(*refs):
        if weighted:
            (s_hbm, d_hbm, als_hbm, ald_hbm, c_hbm, feat_hbm, zer_hbm,
             num_out, den_out, sidx_v, didx_v, rows_v, num_sh,
             sg0, sg1, ss0, ss1,
             als_v, ald_v, c_v, den_v, w_v) = refs
        else:
            (s_hbm, d_hbm, feat_hbm, zer_hbm, num_out,
             sidx_v, didx_v, rows_v, num_sh,
             sg0, sg1, ss0, ss1) = refs
        sems_g = (sg0, sg1)
        sems_s = (ss0, ss1)
        cid = lax.axis_index("c")
        sid = lax.axis_index("s")
        wid = sid * 2 + cid
        pltpu.sync_copy(s_hbm.at[wid], sidx_v)
        pltpu.sync_copy(d_hbm.at[wid], didx_v)
        if weighted:
            pltpu.sync_copy(als_hbm, als_v)
            pltpu.sync_copy(ald_hbm, ald_v)
            pltpu.sync_copy(c_hbm, c_v)

            def zden(i, carry):
                den_v[pl.ds(i * 16, 16)] = jnp.zeros((16,), _F32)
                return carry
            lax.fori_loop(0, (H * NT) // 16, zden, 0)
        pltpu.sync_copy(zer_hbm.at[pl.ds(sid * ROWS, ROWS)],
                        num_sh.at[pl.ds(sid * ROWS, ROWS)])
        plsc.subcore_barrier()

        gd = [None] * NG
        sd = [None] * NG
        gd[0] = pltpu.async_copy(feat_hbm.at[sidx_v.at[0]],
                                 rows_v.at[0], sems_g[0])
        for g in range(NG):
            b = g % 2
            if weighted:
                for j16 in range(GRP // 16):
                    s16 = sidx_v[g, pl.ds(j16 * 16, 16)]
                    d16 = didx_v[g, pl.ds(j16 * 16, 16)]
                    for h in range(H):
                        hoff = jnp.full((16,), h * NT, _I32)
                        a_s = plsc.load_gather(als_v, [s16 + hoff])
                        a_d = plsc.load_gather(ald_v, [d16 + hoff])
                        e = a_s + a_d
                        e = jnp.where(e > 0, e, e * 0.2)
                        w = jnp.exp(e - c_v[h, :])
                        plsc.addupdate_scatter(den_v, [d16 + hoff], w)
                        w_v[h, pl.ds(j16 * 16, 16)] = w
            if g + 1 < NG:
                if g >= 1:
                    sd[g - 1].wait()
                gd[g + 1] = pltpu.async_copy(
                    feat_hbm.at[sidx_v.at[g + 1]],
                    rows_v.at[(g + 1) % 2], sems_g[(g + 1) % 2])
            gd[g].wait()
            if weighted:
                @plsc.parallel_loop(0, GRP, unroll=4)
                def rowscale(j):
                    for h in range(H):
                        wb = plsc.load_gather(
                            w_v, [jnp.full((16,), h, _I32),
                                  jnp.full((16,), j, _I32)])
                        for cc in range(NC16):
                            col = h * OC + cc * 16
                            rows_v[b, j, pl.ds(col, 16)] = (
                                rows_v[b, j, pl.ds(col, 16)] * wb)
            sd[g] = pltpu.async_copy(rows_v.at[b],
                                     num_sh.at[didx_v.at[g]],
                                     sems_s[b], add=True)
        sd[NG - 1].wait()
        if NG >= 2:
            sd[NG - 2].wait()

        plsc.subcore_barrier()
        pltpu.sync_copy(num_sh.at[pl.ds(sid * ROWS, ROWS)],
                        num_out.at[cid, pl.ds(sid * ROWS, ROWS)])
        if weighted:
            pltpu.sync_copy(den_v, den_out.at[wid])

    return functools.partial(
        pl.kernel, out_type=out_type, mesh=mesh, scratch_types=scratch,
        compiler_params=pltpu.CompilerParams(needs_layout_passes=False),
    )(body)


def _edge_agg(s2d, d2d, als, ald, c, feat, zer, H):
    s3 = s2d.reshape(_NW, -1, _GRP)
    d3 = d2d.reshape(_NW, -1, _GRP)
    num_p, den_p = _edge_kernel(s2d.size, feat.shape[0], H, True)(
        s3, d3, als, ald, c, feat, zer)
    return num_p, den_p


def _edge_sum(s2d, d2d, feat, zer):
    s3 = s2d.reshape(_NW, -1, _GRP)
    d3 = d2d.reshape(_NW, -1, _GRP)
    (num_p,) = _edge_kernel(s2d.size, feat.shape[0], 1, False)(
        s3, d3, feat, zer)
    return num_p


def _tc(fn, out_shapes, *args):
    n_in = len(args)

    def body(*refs):
        vals = fn(*[r[...] for r in refs[:n_in]])
        if len(out_shapes) == 1:
            vals = (vals,)
        for r, v in zip(refs[n_in:], vals):
            r[...] = v

    res = pl.pallas_call(
        body,
        out_shape=[jax.ShapeDtypeStruct(s, _F32) for s in out_shapes],
    )(*args)
    return res[0] if len(out_shapes) == 1 else res


def _logits(src_t, dst_t, asrc, adst, H):
    oc = D // H
    als, ald, cs = [], [], []
    for h in range(H):
        a = jnp.sum(src_t[:, h * oc:(h + 1) * oc] * asrc[h][None, :], -1)
        b = jnp.sum(dst_t[:, h * oc:(h + 1) * oc] * adst[h][None, :], -1)
        als.append(a)
        ald.append(b)
        cs.append(jnp.full((16,), jnp.max(a) + jnp.max(b), _F32))
    return (jnp.stack(als), jnp.stack(ald), jnp.stack(cs))


def _gat_pre_same(x1, x2, W, asrc, adst, H):
    tbl = jnp.concatenate([_mm(x1, W), _mm(x2, W)], 0)
    als, ald, c = _logits(tbl, tbl, asrc, adst, H)
    return tbl, als, ald, c


def _gat_pre_merged(he, te, Wia, ia_asrc, ia_adst, Wsrc, Wdst, asrc, adst):
    ia_t = jnp.concatenate([_mm(he, Wia), _mm(te, Wia)], 0)
    src_t = jnp.concatenate([_mm(he, Wsrc), _mm(te, Wsrc)], 0)
    dst_t = jnp.concatenate([_mm(he, Wdst), _mm(te, Wdst)], 0)
    als_i, ald_i, _ = _logits(ia_t, ia_t, ia_asrc, ia_adst, 1)
    als_e, ald_e, _ = _logits(src_t, dst_t, asrc, adst, 1)
    als = jnp.concatenate([als_i, als_e], 1)
    ald = jnp.concatenate([ald_i, ald_e], 1)
    c = jnp.max(als) + jnp.max(ald)
    return (jnp.concatenate([ia_t, src_t], 0), als, ald,
            jnp.full((1, 16), c, _F32))


def _gat_norm(num_p, den_p, bias, H):
    num = jnp.sum(num_p, 0)
    nt = num.shape[0]
    den = jnp.sum(den_p, 0).reshape(H, nt)
    oc = D // H
    cols = [jnp.broadcast_to(den[h][:, None], (nt, oc)) for h in range(H)]
    den_rep = jnp.concatenate(cols, 1)
    return num / (den_rep + 1e-16) + bias[None, :]


def _onehot(batch):
    return (batch[None, :] ==
            lax.broadcasted_iota(_I32, (G, N), 0)).astype(_F32)


def _sag_side(x, nb, attn, batch):
    oh = _onehot(batch)
    m = jnp.max(jnp.where(oh > 0, attn[None, :], -jnp.inf), 1)
    m = jnp.where(jnp.isfinite(m), m, 0.0)
    ex = jnp.exp(attn - _hi(m[None, :], oh)[0])
    den = _hi(oh, ex[:, None])[:, 0]
    score = ex / (_hi(den[None, :], oh)[0] + 1e-16)
    sag = x * score[:, None]
    return _hi(oh, sag)


def _stage_init_fc(h_x, t_x, g, b, W, asrc, adst):
    return _gat_pre_same(_ln(h_x, g, b), _ln(t_x, g, b), W, asrc, adst, H=2)


def _stage_fc_post_merged(num_p, den_p, fc_b, pg_g, pg_b,
                          Wia, ia_asrc, ia_adst, Wsrc, Wdst, asrc, adst):
    out = _gat_norm(num_p, den_p, fc_b, 2)
    he = _elu(_ln(out[:N], pg_g, pg_b))
    te = _elu(_ln(out[N:], pg_g, pg_b))
    return _gat_pre_merged(he, te, Wia, ia_asrc, ia_adst,
                           Wsrc, Wdst, asrc, adst)


def _stage_red(num_m, den_m, ia_b, int_b, red_W, red_b):
    both = _gat_norm(num_m, den_m, jnp.zeros((D,), _F32), 1)
    intra = both[:NS] + ia_b[None, :]
    inter = both[NS:] + int_b[None, :]
    h = _mm(jnp.concatenate([intra[:N], inter[:N]], 1), red_W) + red_b
    t = _mm(jnp.concatenate([intra[N:], inter[N:]], 1), red_W) + red_b
    return jnp.concatenate([h, t], 0), intra, inter


def _sag_pools(hf, nb_p, Wrel, Wroot, brel, h_batch, t_batch):
    nb = jnp.sum(nb_p, 0)
    attn = (_mm(nb, Wrel) + brel + _mm(hf, Wroot))[:, 0]
    h_pool = _sag_side(hf[:N], nb[:N], attn[:N], h_batch)
    t_pool = _sag_side(hf[N:], nb[N:], attn[N:], t_batch)
    return h_pool, t_pool


def _stage_sag_fc(hf, nb_p, Wrel, Wroot, brel, h_batch, t_batch,
                  nn_g, nn_b, W, asrc, adst):
    h_pool, t_pool = _sag_pools(hf, nb_p, Wrel, Wroot, brel,
                                h_batch, t_batch)
    nxt = _elu(_ln(hf, nn_g, nn_b))
    tbl, als, ald, c = _gat_pre_same(nxt[:N], nxt[N:], W, asrc, adst, H=2)
    return h_pool, t_pool, tbl, als, ald, c


def _stage_sag_contrast(hf, nb_p, Wrel, Wroot, brel, h_batch, t_batch,
                        intra, inter, W1, b1, n1w, n1b, W2, b2, n2w, n2b):
    h_pool, t_pool = _sag_pools(hf, nb_p, Wrel, Wroot, brel,
                                h_batch, t_batch)
    ip, ep, zh, zt = _stage_contrast_pre(intra, inter, W1, b1, n1w, n1b,
                                         W2, b2, n2w, n2b)
    return h_pool, t_pool, ip, ep, zh, zt


def _projector(x, W1, b1, n1w, n1b, W2, b2, n2w, n2b):
    h = _mm(x, W1) + b1
    h = h * n1w / jnp.sqrt(1.0 + 1e-5) + n1b
    h = jnp.maximum(h, 0.0)
    h = _mm(h, W2) + b2
    h = h * n2w / jnp.sqrt(1.0 + 1e-5) + n2b
    return h


def _stage_contrast_pre(intra, inter, W1, b1, n1w, n1b, W2, b2, n2w, n2b):
    ip = _normed(_projector(intra, W1, b1, n1w, n1b, W2, b2, n2w, n2b))
    ep = _normed(_projector(inter, W1, b1, n1w, n1b, W2, b2, n2w, n2b))
    zh = _normed(jnp.concatenate([ip[:N], ep[:N]], 0))
    zt = _normed(jnp.concatenate([ip[N:], ep[N:]], 0))
    return ip, ep, zh, zt


def _lse2(zh, zt):
    BLK = 512

    def one(i, zb, zf):
        s = lax.dot_general(zb, zf, (((1,), (1,)), ((), ())),
                            precision=lax.Precision.DEFAULT,
                            preferred_element_type=_F32) * 2.0
        rows = i * BLK + lax.broadcasted_iota(_I32, (BLK, NS), 0)
        cols = lax.broadcasted_iota(_I32, (BLK, NS), 1)
        s = s - jnp.where(rows == cols, 1e9, 0.0)
        m = jnp.max(s, 1)
        return (m + jnp.log(jnp.sum(jnp.exp(s - m[:, None]), 1)))[:, None]

    def body(zhb_ref, zh_ref, ztb_ref, zt_ref, oh_ref, ot_ref):
        i = pl.program_id(0)
        oh_ref[...] = one(i, zhb_ref[...], zh_ref[...])
        ot_ref[...] = one(i, ztb_ref[...], zt_ref[...])

    return pl.pallas_call(
        body,
        grid=(NS // BLK,),
        in_specs=[pl.BlockSpec((BLK, D), lambda i: (i, 0)),
                  pl.BlockSpec((NS, D), lambda i: (0, 0)),
                  pl.BlockSpec((BLK, D), lambda i: (i, 0)),
                  pl.BlockSpec((NS, D), lambda i: (0, 0))],
        out_specs=[pl.BlockSpec((BLK, 1), lambda i: (i, 0)),
                   pl.BlockSpec((BLK, 1), lambda i: (i, 0))],
        out_shape=[jax.ShapeDtypeStruct((NS, 1), _F32),
                   jax.ShapeDtypeStruct((NS, 1), _F32)],
    )(zh, zh, zt, zt)


def _stage_final(lse_h, lse_t, zh, zt, ip, ep, hp1, hp2, tp1, tp2,
                 h_batch, t_batch, rels, cf_W, cf_b, cf_g, cf_bb,
                 fm_W1, fm_b1, fm_g1, fm_bb1, fm_W2, fm_b2, fm_g2, fm_bb2,
                 ca_wq, ca_wk, ca_bias, ca_a, kge3):
    pos_h = jnp.sum(_bf(zh[:N]) * _bf(zh[N:]), -1) * 2.0
    pos_t = jnp.sum(_bf(zt[:N]) * _bf(zt[N:]), -1) * 2.0
    closs = 0.5 * ((jnp.mean(lse_h) - jnp.mean(pos_h)) +
                   (jnp.mean(lse_t) - jnp.mean(pos_t)))

    oh_h = _onehot(h_batch)
    oh_t = _onehot(t_batch)
    hipg = _hi(oh_h, ip[:N])
    hepg = _hi(oh_h, ep[:N])
    tipg = _hi(oh_t, ip[N:])
    tepg = _hi(oh_t, ep[N:])
    h_c = jnp.maximum(_ln(_mm(jnp.concatenate([hipg, hepg], -1), cf_W)
                          + cf_b, cf_g, cf_bb), 0.0)
    t_c = jnp.maximum(_ln(_mm(jnp.concatenate([tipg, tepg], -1), cf_W)
                          + cf_b, cf_g, cf_bb), 0.0)

    def fusion(p, c):
        x = _mm(jnp.concatenate([p, c], -1), fm_W1) + fm_b1
        x = jnp.maximum(_ln(x, fm_g1, fm_bb1), 0.0)
        x = _mm(x, fm_W2) + fm_b2
        x = jnp.maximum(_ln(x, fm_g2, fm_bb2), 0.0)
        return x

    fh = fusion((hp1 + hp2) * 0.5, h_c)
    ft = fusion((tp1 + tp2) * 0.5, t_c)
    fh = _ln(fh, jnp.ones((D,), _F32), jnp.zeros((D,), _F32))
    ft = _ln(ft, jnp.ones((D,), _F32), jnp.zeros((D,), _F32))

    keys = _mm(fh, ca_wk)
    queries = _mm(ft, ca_wq)
    arows = []
    CH = 32
    for i0 in range(0, G, CH):
        e3 = queries[None, :, :] + keys[i0:i0 + CH, None, :] + ca_bias
        t3 = _bf(jnp.tanh(e3))
        arows.append(jnp.sum(t3 * _bf(ca_a)[None, None, :], -1))
    alpha = jnp.concatenate(arows, 0)

    fhn = _bf(_normed(fh))
    ftn = _bf(_normed(ft))
    B = _hi(alpha, ftn)
    A = lax.dot_general(fhn, B, (((0,), (0,)), ((), ())),
                        precision=lax.Precision.HIGHEST,
                        preferred_element_type=_F32)
    nrm = jnp.maximum(jnp.sqrt(jnp.sum(kge3 * kge3, (1, 2))), 1e-12)
    kn = _bf(kge3 / nrm[:, None, None])
    v = jnp.sum(kn * A[None, :, :], (1, 2))
    ohr = (rels[:, None] ==
           lax.broadcasted_iota(_I32, (G, 86), 1)).astype(_F32)
    scores = _hi(ohr, v[:, None])
    return scores, jnp.full((1, 1), 0.0, _F32) + closs


def kernel(h_x, t_x, params, h_edge_index, h_batch, t_edge_index, t_batch,
           b_edge_index, rels):
    P = params
    h_ei = h_edge_index.astype(_I32)
    t_ei = t_edge_index.astype(_I32)
    b_ei = b_edge_index.astype(_I32)
    h_batch = h_batch.astype(_I32)
    t_batch = t_batch.astype(_I32)
    rels = rels.astype(_I32)

    s_in = jnp.concatenate([h_ei[0], t_ei[0] + N]).reshape(-1, _GRP)
    d_in = jnp.concatenate([h_ei[1], t_ei[1] + N]).reshape(-1, _GRP)
    s_b = jnp.concatenate([b_ei[0], b_ei[1] + N])
    d_b = jnp.concatenate([b_ei[1] + N, b_ei[0]])
    s_m = jnp.concatenate([s_in.reshape(-1), s_b + NS]).reshape(-1, _GRP)
    d_m = jnp.concatenate([d_in.reshape(-1), d_b + NS]).reshape(-1, _GRP)
    zer = jnp.zeros((NS, D), _F32)
    zer2 = jnp.zeros((2 * NS, D), _F32)

    B1, B2 = P['blocks']
    pre_shapes = [(NS, D), (2, NS), (2, NS), (2, 16)]
    mrg_shapes = [(2 * NS, D), (1, 2 * NS), (1, 2 * NS), (1, 16)]

    def fc_round(tbl, als, ald, c, Bp):
        num_p, den_p = _edge_agg(s_in, d_in, als.reshape(-1),
                                 ald.reshape(-1), c, tbl, zer, 2)
        tbl, als, ald, c = _tc(
            _stage_fc_post_merged, mrg_shapes,
            num_p, den_p, Bp['fc_b'], Bp['pg_g'], Bp['pg_b'],
            Bp['ia_W'], Bp['ia_asrc'], Bp['ia_adst'],
            Bp['int_Wsrc'], Bp['int_Wdst'],
            Bp['int_asrc'], Bp['int_adst'])
        num_m, den_m = _edge_agg(s_m, d_m, als.reshape(-1),
                                 ald.reshape(-1), c, tbl, zer2, 1)
        hf, intra, inter = _tc(
            _stage_red, [(NS, D), (NS, D), (NS, D)],
            num_m, den_m, Bp['ia_b'], Bp['int_b'],
            Bp['red_W'], Bp['red_b'])
        nb_p = _edge_sum(s_in, d_in, hf, zer)
        return hf, intra, inter, nb_p

    tbl, als, ald, c = _tc(
        _stage_init_fc, pre_shapes,
        h_x, t_x, P['init_g'], P['init_b'],
        B1['fc_W'], B1['fc_asrc'], B1['fc_adst'])
    hf, intra, inter, nb_p = fc_round(tbl, als, ald, c, B1)

    hp1, tp1, tbl, als, ald, c = _tc(
        _stage_sag_fc, [(G, D), (G, D)] + pre_shapes,
        hf, nb_p, B1['sag_Wrel'], B1['sag_Wroot'], B1['sag_brel'],
        h_batch, t_batch, B1['nn_g'], B1['nn_b'],
        B2['fc_W'], B2['fc_asrc'], B2['fc_adst'])
    hf, intra, inter, nb_p = fc_round(tbl, als, ald, c, B2)

    hp2, tp2, ip, ep, zh, zt = _tc(
        _stage_sag_contrast,
        [(G, D), (G, D), (NS, D), (NS, D), (NS, D), (NS, D)],
        hf, nb_p, B2['sag_Wrel'], B2['sag_Wroot'], B2['sag_brel'],
        h_batch, t_batch, intra, inter,
        P['cm_W1'], P['cm_b1'], P['cm_bn1w'], P['cm_bn1b'],
        P['cm_W2'], P['cm_b2'], P['cm_bn2w'], P['cm_bn2b'])
    pools = [(hp1, tp1), (hp2, tp2)]

    lse_h, lse_t = _lse2(zh, zt)

    kge3 = P['kge_rel'].reshape(86, D, D)
    scores, closs = _tc(
        _stage_final, [(G, 1), (1, 1)],
        lse_h, lse_t, zh, zt, ip, ep,
        pools[0][0], pools[1][0], pools[0][1], pools[1][1],
        h_batch, t_batch, rels,
        P['cf_W'], P['cf_b'], P['cf_g'], P['cf_bb'],
        P['fm_W1'], P['fm_b1'], P['fm_g1'], P['fm_bb1'],
        P['fm_W2'], P['fm_b2'], P['fm_g2'], P['fm_bb2'],
        P['ca_wq'], P['ca_wk'], P['ca_bias'], P['ca_a'], kge3)

    return scores.reshape(G), closs.reshape(())


# speedup vs baseline: 23.0396x; 1.0182x over previous
import functools

import jax
import jax.numpy as jnp
from jax import lax
from jax.experimental import pallas as pl
from jax.experimental.pallas import tpu as pltpu
from jax.experimental.pallas import tpu_sc as plsc

N = 2048
NS = 2 * N
D = 128
G = 256
_NW = 32
_GRP = 128

_F32 = jnp.float32
_I32 = jnp.int32


def _hi(a, b):
    return jnp.matmul(a, b, precision=lax.Precision.HIGHEST,
                      preferred_element_type=_F32)


def _mm(a, b):
    return jnp.matmul(a, b, precision=lax.Precision.DEFAULT,
                      preferred_element_type=_F32)


def _bf(x):
    return x.astype(jnp.bfloat16).astype(_F32)


def _ln(x, g, b, eps=1e-5):
    mu = jnp.mean(x, -1, keepdims=True)
    v = jnp.mean((x - mu) ** 2, -1, keepdims=True)
    return (x - mu) / jnp.sqrt(v + eps) * g + b


def _elu(x):
    return jnp.where(x > 0, x, jnp.exp(jnp.minimum(x, 0.0)) - 1.0)


def _normed(x):
    n = jnp.sqrt(jnp.sum(x * x, -1, keepdims=True))
    return x / jnp.maximum(n, 1e-12)


@functools.cache
def _edge_kernel(E, NT, H, weighted):
    GRP = 128
    EW = E // _NW
    NG = EW // GRP
    ROWS = NT // 16
    OC = D // H
    NC16 = OC // 16
    mesh = plsc.VectorSubcoreMesh(core_axis_name="c", subcore_axis_name="s",
                                  num_cores=2, num_subcores=16)

    RING = 2 if NT > NS else min(3, NG)
    out_type = [jax.ShapeDtypeStruct((2, NT, D), _F32)]
    scratch = [
        pltpu.VMEM((NG, GRP), _I32),
        pltpu.VMEM((NG, GRP), _I32),
        pltpu.VMEM((RING, GRP, D), _F32),
        pltpu.VMEM_SHARED((NT, D), _F32),
    ] + [pltpu.SemaphoreType.DMA] * (2 * RING)
    if weighted:
        out_type.append(jax.ShapeDtypeStruct((_NW, H * NT), _F32))
        scratch += [
            pltpu.VMEM((H * NT,), _F32),
            pltpu.VMEM((H * NT,), _F32),
            pltpu.VMEM((H, 16), _F32),
            pltpu.VMEM((H * NT,), _F32),
            pltpu.VMEM((H, GRP), _F32),
        ]

    def body(*refs):
        if weighted:
            (s_hbm, d_hbm, als_hbm, ald_hbm, c_hbm, feat_hbm, zer_hbm,
             zd_hbm, num_out, den_out,
             sidx_v, didx_v, rows_v, num_sh) = refs[:14]
            sems = refs[14:14 + 2 * RING]
            als_v, ald_v, c_v, den_v, w_v = refs[14 + 2 * RING:]
        else:
            (s_hbm, d_hbm, feat_hbm, zer_hbm, num_out,
             sidx_v, didx_v, rows_v, num_sh) = refs[:9]
            sems = refs[9:9 + 2 * RING]
        sems_g = sems[:RING]
        sems_s = sems[RING:]
        cid = lax.axis_index("c")
        sid = lax.axis_index("s")
        wid = sid * 2 + cid
        pltpu.sync_copy(s_hbm.at[wid], sidx_v)
        pltpu.sync_copy(d_hbm.at[wid], didx_v)

        gd = [None] * NG
        sd = [None] * NG
        for r in range(min(RING - 1, NG)):
            gd[r] = pltpu.async_copy(feat_hbm.at[sidx_v.at[r]],
                                     rows_v.at[r], sems_g[r])

        if weighted:
            pltpu.sync_copy(als_hbm, als_v)
            pltpu.sync_copy(ald_hbm, ald_v)
            pltpu.sync_copy(c_hbm, c_v)
            pltpu.sync_copy(zd_hbm, den_v)
        pltpu.sync_copy(zer_hbm.at[pl.ds(sid * ROWS, ROWS)],
                        num_sh.at[pl.ds(sid * ROWS, ROWS)])
        plsc.subcore_barrier()

        waited = [False] * NG

        def wait_sd(i):
            if 0 <= i < NG and sd[i] is not None and not waited[i]:
                sd[i].wait()
                waited[i] = True

        for g in range(NG):
            b = g % RING
            wait_sd(g - 1)
            nxt = g + RING - 1
            if nxt < NG and gd[nxt] is None:
                gd[nxt] = pltpu.async_copy(feat_hbm.at[sidx_v.at[nxt]],
                                           rows_v.at[nxt % RING],
                                           sems_g[nxt % RING])
            if weighted:
                for j16 in range(GRP // 16):
                    s16 = sidx_v[g, pl.ds(j16 * 16, 16)]
                    d16 = didx_v[g, pl.ds(j16 * 16, 16)]
                    for h in range(H):
                        hoff = jnp.full((16,), h * NT, _I32)
                        a_s = plsc.load_gather(als_v, [s16 + hoff])
                        a_d = plsc.load_gather(ald_v, [d16 + hoff])
                        e = a_s + a_d
                        e = jnp.where(e > 0, e, e * 0.2)
                        w = jnp.exp(e - c_v[h, :])
                        plsc.addupdate_scatter(den_v, [d16 + hoff], w)
                        w_v[h, pl.ds(j16 * 16, 16)] = w
            gd[g].wait()
            if weighted:
                @plsc.parallel_loop(0, GRP, unroll=4)
                def rowscale(j):
                    for h in range(H):
                        wb = plsc.load_gather(
                            w_v, [jnp.full((16,), h, _I32),
                                  jnp.full((16,), j, _I32)])
                        for cc in range(NC16):
                            col = h * OC + cc * 16
                            rows_v[b, j, pl.ds(col, 16)] = (
                                rows_v[b, j, pl.ds(col, 16)] * wb)
            sd[g] = pltpu.async_copy(rows_v.at[b],
                                     num_sh.at[didx_v.at[g]],
                                     sems_s[b], add=True)
        for g in range(max(0, NG - 2), NG):
            wait_sd(g)

        plsc.subcore_barrier()
        pltpu.sync_copy(num_sh.at[pl.ds(sid * ROWS, ROWS)],
                        num_out.at[cid, pl.ds(sid * ROWS, ROWS)])
        if weighted:
            pltpu.sync_copy(den_v, den_out.at[wid])

    return functools.partial(
        pl.kernel, out_type=out_type, mesh=mesh, scratch_types=scratch,
        compiler_params=pltpu.CompilerParams(needs_layout_passes=False),
    )(body)


def _edge_agg(s2d, d2d, als, ald, c, feat, zer, H):
    s3 = s2d.reshape(_NW, -1, _GRP)
    d3 = d2d.reshape(_NW, -1, _GRP)
    zd = jnp.zeros((H * feat.shape[0],), _F32)
    num_p, den_p = _edge_kernel(s2d.size, feat.shape[0], H, True)(
        s3, d3, als, ald, c, feat, zer, zd)
    return num_p, den_p


def _edge_sum(s2d, d2d, feat, zer):
    s3 = s2d.reshape(_NW, -1, _GRP)
    d3 = d2d.reshape(_NW, -1, _GRP)
    (num_p,) = _edge_kernel(s2d.size, feat.shape[0], 1, False)(
        s3, d3, feat, zer)
    return num_p


def _tc(fn, out_shapes, *args):
    n_in = len(args)

    def body(*refs):
        vals = fn(*[r[...] for r in refs[:n_in]])
        if len(out_shapes) == 1:
            vals = (vals,)
        for r, v in zip(refs[n_in:], vals):
            r[...] = v

    res = pl.pallas_call(
        body,
        out_shape=[jax.ShapeDtypeStruct(s, _F32) for s in out_shapes],
    )(*args)
    return res[0] if len(out_shapes) == 1 else res


def _logits(src_t, dst_t, asrc, adst, H):
    oc = D // H
    als, ald, cs = [], [], []
    for h in range(H):
        a = jnp.sum(src_t[:, h * oc:(h + 1) * oc] * asrc[h][None, :], -1)
        b = jnp.sum(dst_t[:, h * oc:(h + 1) * oc] * adst[h][None, :], -1)
        als.append(a)
        ald.append(b)
        cs.append(jnp.full((16,), jnp.max(a) + jnp.max(b), _F32))
    return (jnp.stack(als), jnp.stack(ald), jnp.stack(cs))


def _gat_pre_same(x1, x2, W, asrc, adst, H):
    tbl = jnp.concatenate([_mm(x1, W), _mm(x2, W)], 0)
    als, ald, c = _logits(tbl, tbl, asrc, adst, H)
    return tbl, als, ald, c


def _gat_pre_merged(he, te, Wia, ia_asrc, ia_adst, Wsrc, Wdst, asrc, adst):
    ia_t = jnp.concatenate([_mm(he, Wia), _mm(te, Wia)], 0)
    src_t = jnp.concatenate([_mm(he, Wsrc), _mm(te, Wsrc)], 0)
    dst_t = jnp.concatenate([_mm(he, Wdst), _mm(te, Wdst)], 0)
    als_i, ald_i, _ = _logits(ia_t, ia_t, ia_asrc, ia_adst, 1)
    als_e, ald_e, _ = _logits(src_t, dst_t, asrc, adst, 1)
    als = jnp.concatenate([als_i, als_e], 1)
    ald = jnp.concatenate([ald_i, ald_e], 1)
    c = jnp.max(als) + jnp.max(ald)
    return (jnp.concatenate([ia_t, src_t], 0), als, ald,
            jnp.full((1, 16), c, _F32))


def _gat_norm(num_p, den_p, bias, H):
    num = jnp.sum(num_p, 0)
    nt = num.shape[0]
    den = jnp.sum(den_p, 0).reshape(H, nt)
    oc = D // H
    cols = [jnp.broadcast_to(den[h][:, None], (nt, oc)) for h in range(H)]
    den_rep = jnp.concatenate(cols, 1)
    return num / (den_rep + 1e-16) + bias[None, :]


def _onehot(batch):
    return (batch[None, :] ==
            lax.broadcasted_iota(_I32, (G, N), 0)).astype(_F32)


def _sag_side(x, nb, attn, batch):
    oh = _onehot(batch)
    m = jnp.max(jnp.where(oh > 0, attn[None, :], -jnp.inf), 1)
    m = jnp.where(jnp.isfinite(m), m, 0.0)
    ex = jnp.exp(attn - _hi(m[None, :], oh)[0])
    den = _hi(oh, ex[:, None])[:, 0]
    score = ex / (_hi(den[None, :], oh)[0] + 1e-16)
    sag = x * score[:, None]
    return _hi(oh, sag)


def _stage_init_fc(h_x, t_x, g, b, W, asrc, adst):
    return _gat_pre_same(_ln(h_x, g, b), _ln(t_x, g, b), W, asrc, adst, H=2)


def _stage_fc_post_merged(num_p, den_p, fc_b, pg_g, pg_b,
                          Wia, ia_asrc, ia_adst, Wsrc, Wdst, asrc, adst):
    out = _gat_norm(num_p, den_p, fc_b, 2)
    he = _elu(_ln(out[:N], pg_g, pg_b))
    te = _elu(_ln(out[N:], pg_g, pg_b))
    return _gat_pre_merged(he, te, Wia, ia_asrc, ia_adst,
                           Wsrc, Wdst, asrc, adst)


def _stage_red(num_m, den_m, ia_b, int_b, red_W, red_b):
    both = _gat_norm(num_m, den_m, jnp.zeros((D,), _F32), 1)
    intra = both[:NS] + ia_b[None, :]
    inter = both[NS:] + int_b[None, :]
    h = _mm(jnp.concatenate([intra[:N], inter[:N]], 1), red_W) + red_b
    t = _mm(jnp.concatenate([intra[N:], inter[N:]], 1), red_W) + red_b
    return jnp.concatenate([h, t], 0), intra, inter


def _sag_pools(hf, nb_p, Wrel, Wroot, brel, h_batch, t_batch):
    nb = jnp.sum(nb_p, 0)
    attn = (_mm(nb, Wrel) + brel + _mm(hf, Wroot))[:, 0]
    h_pool = _sag_side(hf[:N], nb[:N], attn[:N], h_batch)
    t_pool = _sag_side(hf[N:], nb[N:], attn[N:], t_batch)
    return h_pool, t_pool


def _stage_sag_fc(hf, nb_p, Wrel, Wroot, brel, h_batch, t_batch,
                  nn_g, nn_b, W, asrc, adst):
    h_pool, t_pool = _sag_pools(hf, nb_p, Wrel, Wroot, brel,
                                h_batch, t_batch)
    nxt = _elu(_ln(hf, nn_g, nn_b))
    tbl, als, ald, c = _gat_pre_same(nxt[:N], nxt[N:], W, asrc, adst, H=2)
    return h_pool, t_pool, tbl, als, ald, c


def _stage_sag_contrast(hf, nb_p, Wrel, Wroot, brel, h_batch, t_batch,
                        intra, inter, W1, b1, n1w, n1b, W2, b2, n2w, n2b):
    h_pool, t_pool = _sag_pools(hf, nb_p, Wrel, Wroot, brel,
                                h_batch, t_batch)
    ip, ep, zh, zt = _stage_contrast_pre(intra, inter, W1, b1, n1w, n1b,
                                         W2, b2, n2w, n2b)
    return h_pool, t_pool, ip, ep, zh, zt


def _projector(x, W1, b1, n1w, n1b, W2, b2, n2w, n2b):
    h = _mm(x, W1) + b1
    h = h * n1w / jnp.sqrt(1.0 + 1e-5) + n1b
    h = jnp.maximum(h, 0.0)
    h = _mm(h, W2) + b2
    h = h * n2w / jnp.sqrt(1.0 + 1e-5) + n2b
    return h


def _stage_contrast_pre(intra, inter, W1, b1, n1w, n1b, W2, b2, n2w, n2b):
    ip = _normed(_projector(intra, W1, b1, n1w, n1b, W2, b2, n2w, n2b))
    ep = _normed(_projector(inter, W1, b1, n1w, n1b, W2, b2, n2w, n2b))
    zh = _normed(jnp.concatenate([ip[:N], ep[:N]], 0))
    zt = _normed(jnp.concatenate([ip[N:], ep[N:]], 0))
    return ip, ep, zh, zt


def _lse2(zh, zt):
    BLK = 512

    def one(i, zb, zf):
        s = lax.dot_general(zb, zf, (((1,), (1,)), ((), ())),
                            precision=lax.Precision.DEFAULT,
                            preferred_element_type=_F32) * 2.0
        rows = i * BLK + lax.broadcasted_iota(_I32, (BLK, NS), 0)
        cols = lax.broadcasted_iota(_I32, (BLK, NS), 1)
        s = s - jnp.where(rows == cols, 1e9, 0.0)
        m = jnp.max(s, 1)
        return (m + jnp.log(jnp.sum(jnp.exp(s - m[:, None]), 1)))[:, None]

    def body(zhb_ref, zh_ref, ztb_ref, zt_ref, oh_ref, ot_ref):
        i = pl.program_id(0)
        oh_ref[...] = one(i, zhb_ref[...], zh_ref[...])
        ot_ref[...] = one(i, ztb_ref[...], zt_ref[...])

    return pl.pallas_call(
        body,
        grid=(NS // BLK,),
        in_specs=[pl.BlockSpec((BLK, D), lambda i: (i, 0)),
                  pl.BlockSpec((NS, D), lambda i: (0, 0)),
                  pl.BlockSpec((BLK, D), lambda i: (i, 0)),
                  pl.BlockSpec((NS, D), lambda i: (0, 0))],
        out_specs=[pl.BlockSpec((BLK, 1), lambda i: (i, 0)),
                   pl.BlockSpec((BLK, 1), lambda i: (i, 0))],
        out_shape=[jax.ShapeDtypeStruct((NS, 1), _F32),
                   jax.ShapeDtypeStruct((NS, 1), _F32)],
    )(zh, zh, zt, zt)


def _stage_final(lse_h, lse_t, zh, zt, ip, ep, hp1, hp2, tp1, tp2,
                 h_batch, t_batch, rels, cf_W, cf_b, cf_g, cf_bb,
                 fm_W1, fm_b1, fm_g1, fm_bb1, fm_W2, fm_b2, fm_g2, fm_bb2,
                 ca_wq, ca_wk, ca_bias, ca_a, kge3):
    pos_h = jnp.sum(_bf(zh[:N]) * _bf(zh[N:]), -1) * 2.0
    pos_t = jnp.sum(_bf(zt[:N]) * _bf(zt[N:]), -1) * 2.0
    closs = 0.5 * ((jnp.mean(lse_h) - jnp.mean(pos_h)) +
                   (jnp.mean(lse_t) - jnp.mean(pos_t)))

    oh_h = _onehot(h_batch)
    oh_t = _onehot(t_batch)
    hipg = _hi(oh_h, ip[:N])
    hepg = _hi(oh_h, ep[:N])
    tipg = _hi(oh_t, ip[N:])
    tepg = _hi(oh_t, ep[N:])
    h_c = jnp.maximum(_ln(_mm(jnp.concatenate([hipg, hepg], -1), cf_W)
                          + cf_b, cf_g, cf_bb), 0.0)
    t_c = jnp.maximum(_ln(_mm(jnp.concatenate([tipg, tepg], -1), cf_W)
                          + cf_b, cf_g, cf_bb), 0.0)

    def fusion(p, c):
        x = _mm(jnp.concatenate([p, c], -1), fm_W1) + fm_b1
        x = jnp.maximum(_ln(x, fm_g1, fm_bb1), 0.0)
        x = _mm(x, fm_W2) + fm_b2
        x = jnp.maximum(_ln(x, fm_g2, fm_bb2), 0.0)
        return x

    fh = fusion((hp1 + hp2) * 0.5, h_c)
    ft = fusion((tp1 + tp2) * 0.5, t_c)
    fh = _ln(fh, jnp.ones((D,), _F32), jnp.zeros((D,), _F32))
    ft = _ln(ft, jnp.ones((D,), _F32), jnp.zeros((D,), _F32))

    keys = _mm(fh, ca_wk)
    queries = _mm(ft, ca_wq)
    arows = []
    CH = 32
    for i0 in range(0, G, CH):
        e3 = queries[None, :, :] + keys[i0:i0 + CH, None, :] + ca_bias
        t3 = _bf(jnp.tanh(e3))
        arows.append(jnp.sum(t3 * _bf(ca_a)[None, None, :], -1))
    alpha = jnp.concatenate(arows, 0)

    fhn = _bf(_normed(fh))
    ftn = _bf(_normed(ft))
    B = _hi(alpha, ftn)
    A = lax.dot_general(fhn, B, (((0,), (0,)), ((), ())),
                        precision=lax.Precision.HIGHEST,
                        preferred_element_type=_F32)
    nrm = jnp.maximum(jnp.sqrt(jnp.sum(kge3 * kge3, (1, 2))), 1e-12)
    kn = _bf(kge3 / nrm[:, None, None])
    v = jnp.sum(kn * A[None, :, :], (1, 2))
    ohr = (rels[:, None] ==
           lax.broadcasted_iota(_I32, (G, 86), 1)).astype(_F32)
    scores = _hi(ohr, v[:, None])
    return scores, jnp.full((1, 1), 0.0, _F32) + closs


def kernel(h_x, t_x, params, h_edge_index, h_batch, t_edge_index, t_batch,
           b_edge_index, rels):
    P = params
    h_ei = h_edge_index.astype(_I32)
    t_ei = t_edge_index.astype(_I32)
    b_ei = b_edge_index.astype(_I32)
    h_batch = h_batch.astype(_I32)
    t_batch = t_batch.astype(_I32)
    rels = rels.astype(_I32)

    s_in = jnp.concatenate([h_ei[0], t_ei[0] + N]).reshape(-1, _GRP)
    d_in = jnp.concatenate([h_ei[1], t_ei[1] + N]).reshape(-1, _GRP)
    s_b = jnp.concatenate([b_ei[0], b_ei[1] + N])
    d_b = jnp.concatenate([b_ei[1] + N, b_ei[0]])
    s_m = jnp.concatenate([s_in.reshape(-1), s_b + NS]).reshape(-1, _GRP)
    d_m = jnp.concatenate([d_in.reshape(-1), d_b + NS]).reshape(-1, _GRP)
    zer = jnp.zeros((NS, D), _F32)
    zer2 = jnp.zeros((2 * NS, D), _F32)

    B1, B2 = P['blocks']
    pre_shapes = [(NS, D), (2, NS), (2, NS), (2, 16)]
    mrg_shapes = [(2 * NS, D), (1, 2 * NS), (1, 2 * NS), (1, 16)]

    def fc_round(tbl, als, ald, c, Bp):
        num_p, den_p = _edge_agg(s_in, d_in, als.reshape(-1),
                                 ald.reshape(-1), c, tbl, zer, 2)
        tbl, als, ald, c = _tc(
            _stage_fc_post_merged, mrg_shapes,
            num_p, den_p, Bp['fc_b'], Bp['pg_g'], Bp['pg_b'],
            Bp['ia_W'], Bp['ia_asrc'], Bp['ia_adst'],
            Bp['int_Wsrc'], Bp['int_Wdst'],
            Bp['int_asrc'], Bp['int_adst'])
        num_m, den_m = _edge_agg(s_m, d_m, als.reshape(-1),
                                 ald.reshape(-1), c, tbl, zer2, 1)
        hf, intra, inter = _tc(
            _stage_red, [(NS, D), (NS, D), (NS, D)],
            num_m, den_m, Bp['ia_b'], Bp['int_b'],
            Bp['red_W'], Bp['red_b'])
        nb_p = _edge_sum(s_in, d_in, hf, zer)
        return hf, intra, inter, nb_p

    tbl, als, ald, c = _tc(
        _stage_init_fc, pre_shapes,
        h_x, t_x, P['init_g'], P['init_b'],
        B1['fc_W'], B1['fc_asrc'], B1['fc_adst'])
    hf, intra, inter, nb_p = fc_round(tbl, als, ald, c, B1)

    hp1, tp1, tbl, als, ald, c = _tc(
        _stage_sag_fc, [(G, D), (G, D)] + pre_shapes,
        hf, nb_p, B1['sag_Wrel'], B1['sag_Wroot'], B1['sag_brel'],
        h_batch, t_batch, B1['nn_g'], B1['nn_b'],
        B2['fc_W'], B2['fc_asrc'], B2['fc_adst'])
    hf, intra, inter, nb_p = fc_round(tbl, als, ald, c, B2)

    hp2, tp2, ip, ep, zh, zt = _tc(
        _stage_sag_contrast,
        [(G, D), (G, D), (NS, D), (NS, D), (NS, D), (NS, D)],
        hf, nb_p, B2['sag_Wrel'], B2['sag_Wroot'], B2['sag_brel'],
        h_batch, t_batch, intra, inter,
        P['cm_W1'], P['cm_b1'], P['cm_bn1w'], P['cm_bn1b'],
        P['cm_W2'], P['cm_b2'], P['cm_bn2w'], P['cm_bn2b'])
    pools = [(hp1, tp1), (hp2, tp2)]

    lse_h, lse_t = _lse2(zh, zt)

    kge3 = P['kge_rel'].reshape(86, D, D)
    scores, closs = _tc(
        _stage_final, [(G, 1), (1, 1)],
        lse_h, lse_t, zh, zt, ip, ep,
        pools[0][0], pools[1][0], pools[0][1], pools[1][1],
        h_batch, t_batch, rels,
        P['cf_W'], P['cf_b'], P['cf_g'], P['cf_bb'],
        P['fm_W1'], P['fm_b1'], P['fm_g1'], P['fm_bb1'],
        P['fm_W2'], P['fm_b2'], P['fm_g2'], P['fm_bb2'],
        P['ca_wq'], P['ca_wk'], P['ca_bias'], P['ca_a'], kge3)

    return scores.reshape(G), closs.reshape(())
